# baseline (speedup 1.0000x reference)
import functools

import jax
import jax.numpy as jnp
from jax import lax
from jax.experimental import pallas as pl
from jax.experimental.pallas import tpu as pltpu
from jax.experimental.pallas import tpu_sc as plsc

BATCH = 16384
D = 64
NC = 2
NS = 16
NW = NC * NS
BPW = BATCH // NW
CHUNK = 128
NCHUNK = BPW // CHUNK


def _mf_body(users_hbm, items_hbm, uf_hbm, if_hbm, out_hbm,
             uidx, iidx, urows, irows, outv, sem):
    wid = lax.axis_index("s") * NC + lax.axis_index("c")
    base = wid * BPW

    pltpu.sync_copy(users_hbm.at[wid], uidx)
    pltpu.sync_copy(items_hbm.at[wid], iidx)

    copies = []
    for j in range(NCHUNK):
        copies.append(pltpu.async_copy(
            uf_hbm.at[uidx.at[j]], urows.at[pl.ds(j * CHUNK, CHUNK)], sem))
        copies.append(pltpu.async_copy(
            if_hbm.at[iidx.at[j]], irows.at[pl.ds(j * CHUNK, CHUNK)], sem))
    for c in copies:
        c.wait()

    lane = lax.iota(jnp.int32, 16)

    def group(g, carry):
        rows = g * 16 + lane
        colv = lane * 0
        accs = [jnp.zeros((16,), jnp.float32) for _ in range(4)]
        for c in range(D):
            u = plsc.load_gather(urows, [rows, colv])
            v = plsc.load_gather(irows, [rows, colv])
            accs[c % 4] = accs[c % 4] + u * v
            colv = colv + 1
        acc = (accs[0] + accs[1]) + (accs[2] + accs[3])
        outv[pl.ds(g * 16, 16)] = acc
        return carry

    lax.fori_loop(0, BPW // 16, group, 0)

    pltpu.sync_copy(outv, out_hbm.at[pl.ds(base, BPW)])


@jax.jit
def _mf(users3d, items3d, user_factors, item_factors):
    mesh = plsc.VectorSubcoreMesh(core_axis_name="c", subcore_axis_name="s")
    kern = functools.partial(
        pl.kernel,
        mesh=mesh,
        compiler_params=pltpu.CompilerParams(
            needs_layout_passes=False, use_tc_tiling_on_sc=False),
        out_type=jax.ShapeDtypeStruct((BATCH,), jnp.float32),
        scratch_types=[
            pltpu.VMEM((NCHUNK, CHUNK), jnp.int32),
            pltpu.VMEM((NCHUNK, CHUNK), jnp.int32),
            pltpu.VMEM((BPW, D), jnp.float32),
            pltpu.VMEM((BPW, D), jnp.float32),
            pltpu.VMEM((BPW,), jnp.float32),
            pltpu.SemaphoreType.DMA,
        ],
    )(_mf_body)
    return kern(users3d, items3d, user_factors, item_factors)


def kernel(data, user_factors, item_factors):
    users3d = data[:, 0].astype(jnp.int32).reshape(NW, NCHUNK, CHUNK)
    items3d = data[:, 1].astype(jnp.int32).reshape(NW, NCHUNK, CHUNK)
    return _mf(users3d, items3d, user_factors, item_factors)
